# baseline (speedup 1.0000x reference)
import functools

import jax
import jax.numpy as jnp
from jax import lax
from jax.experimental import pallas as pl
from jax.experimental.pallas import tpu as pltpu
from jax.experimental.pallas import tpu_sc as plsc

B, NF, L = 4096, 7, 50
V, D = 1000000, 31
OUT = 3
DP = 32
N_ROWS = B * NF * L
NC, NS = 2, 16
NW = NC * NS
ROWS_PER_W = N_ROWS // NW
CHUNK = 128
NBUF = 4
N_GROUPS = ROWS_PER_W // (CHUNK * NBUF)
VB = 128
NVB = V // VB
VTAIL = V - NVB * VB
BLOCKS_PER_W = (NVB + NW - 1) // NW


def _transpose_block(src_v, dst_v, iota32, nv16):
    for g in range(nv16):
        for d in range(D):
            val = src_v[d, pl.ds(g * 16, 16)]
            plsc.store_scatter(dst_v, [iota32 + (g * 16 * DP + d)], val)


def _sc_relayout_body(tt_hbm, lin_hbm, src_v, dst_v, src_t, dst_t):
    wid = lax.axis_index("s") * NC + lax.axis_index("c")
    iota32 = lax.iota(jnp.int32, 16) * DP

    for f in range(NF):
        def blk(k, carry):
            vb = k * NW + wid

            @pl.when(vb < NVB)
            def _():
                v0 = vb * VB
                pltpu.sync_copy(tt_hbm.at[f, :, pl.ds(v0, VB)], src_v)
                _transpose_block(src_v, dst_v, iota32, VB // 16)
                pltpu.sync_copy(dst_v,
                                lin_hbm.at[pl.ds((f * V + v0) * DP, VB * DP)])
            return carry

        lax.fori_loop(0, BLOCKS_PER_W, blk, 0)

    for f in range(NF):
        @pl.when(wid == f)
        def _():
            v0 = NVB * VB
            pltpu.sync_copy(tt_hbm.at[f, :, pl.ds(v0, VTAIL)], src_t)
            _transpose_block(src_t, dst_t, iota32, VTAIL // 16)
            pltpu.sync_copy(dst_t,
                            lin_hbm.at[pl.ds((f * V + v0) * DP, VTAIL * DP)])


@functools.cache
def _sc_relayout():
    return pl.kernel(
        _sc_relayout_body,
        out_type=jax.ShapeDtypeStruct((NF * V * DP,), jnp.float32),
        mesh=plsc.VectorSubcoreMesh(core_axis_name="c", subcore_axis_name="s",
                                    num_cores=NC, num_subcores=NS),
        scratch_types=[
            pltpu.VMEM((D, VB), jnp.float32),
            pltpu.VMEM((VB * DP,), jnp.float32),
            pltpu.VMEM((D, VTAIL), jnp.float32),
            pltpu.VMEM((VTAIL * DP,), jnp.float32),
        ],
        compiler_params=pltpu.CompilerParams(use_tc_tiling_on_sc=True,
                                             needs_layout_passes=False),
    )


def _sc_gather_body(table_hbm, idx_hbm, out_hbm, idx_v, rows_v, sems):
    wid = lax.axis_index("s") * NC + lax.axis_index("c")
    base = wid * ROWS_PER_W
    total = N_ROWS

    def group(g, carry):
        row0 = base + g * (CHUNK * NBUF)
        for j in range(NBUF):
            @pl.when(row0 + j * CHUNK < base + ROWS_PER_W)
            def _():
                pltpu.sync_copy(idx_hbm.at[pl.ds(row0 + j * CHUNK, CHUNK)],
                                idx_v.at[j])
                pltpu.async_copy(table_hbm.at[idx_v.at[j]], rows_v.at[j],
                                 sems.at[j])
        for j in range(NBUF):
            @pl.when(row0 + j * CHUNK < base + ROWS_PER_W)
            def _():
                pltpu.make_async_copy(table_hbm.at[idx_v.at[j]], rows_v.at[j],
                                      sems.at[j]).wait()
                pltpu.sync_copy(rows_v.at[j],
                                out_hbm.at[pl.ds(row0 + j * CHUNK, CHUNK)])
        return carry

    lax.fori_loop(0, (ROWS_PER_W + CHUNK * NBUF - 1) // (CHUNK * NBUF),
                  group, 0)


@functools.cache
def _sc_gather():
    return pl.kernel(
        _sc_gather_body,
        out_type=jax.ShapeDtypeStruct((N_ROWS, DP), jnp.float32),
        mesh=plsc.VectorSubcoreMesh(core_axis_name="c", subcore_axis_name="s",
                                    num_cores=NC, num_subcores=NS),
        scratch_types=[
            pltpu.VMEM((NBUF, CHUNK), jnp.int32),
            pltpu.VMEM((NBUF, CHUNK, DP), jnp.float32),
            pltpu.SemaphoreType.DMA((NBUF,)),
        ],
        compiler_params=pltpu.CompilerParams(use_tc_tiling_on_sc=False),
    )


def _attn_encode_body(emb_ref, w_ref, encw_ref, encb_ref, out_ref):
    i = pl.program_id(0)
    xt = jnp.tanh(emb_ref[0][:, :D])
    e = jnp.sum(xt * w_ref[0, 0], axis=1, keepdims=True)
    p = jnp.exp(e - jnp.max(e))
    a = p * (1.0 / jnp.sum(p))
    contrib = jnp.dot(xt * a, encw_ref[0],
                      preferred_element_type=jnp.float32)
    prev = jnp.where(i == 0, 0.0, out_ref[...])
    tot = prev + contrib
    is_last = i == NF * L - 1
    out_ref[...] = jnp.where(is_last,
                             jnp.maximum(tot + encb_ref[...], 0.0), tot)


def kernel(x, tables, attn_w, attn_b, enc_w, enc_b):
    del attn_b
    tt = jnp.transpose(tables, (0, 2, 1))
    lin = _sc_relayout()(tt)
    table_flat = lin.reshape(NF * V, DP)

    idx = (jnp.transpose(x, (1, 2, 0))
           + (jnp.arange(NF, dtype=jnp.int32) * V)[:, None, None])
    idx_flat = idx.reshape(N_ROWS)

    emb = _sc_gather()(table_flat, idx_flat)
    emb3 = emb.reshape(NF * L, B, DP)

    out = pl.pallas_call(
        _attn_encode_body,
        grid=(NF * L,),
        in_specs=[
            pl.BlockSpec((1, B, DP), lambda i: (i, 0, 0)),
            pl.BlockSpec((1, 1, D), lambda i: (i // L, 0, 0)),
            pl.BlockSpec((1, D, OUT), lambda i: (i // L, 0, 0)),
            pl.BlockSpec((1, OUT), lambda i: (0, 0)),
        ],
        out_specs=pl.BlockSpec((B, OUT), lambda i: (0, 0)),
        out_shape=jax.ShapeDtypeStruct((B, OUT), jnp.float32),
    )(emb3, attn_w.reshape(NF, 1, D), enc_w.reshape(NF, D, OUT),
      enc_b.reshape(1, OUT))
    return out


# speedup vs baseline: 1.3226x; 1.3226x over previous
import functools

import jax
import jax.numpy as jnp
from jax import lax
from jax.experimental import pallas as pl
from jax.experimental.pallas import tpu as pltpu
from jax.experimental.pallas import tpu_sc as plsc

B, NF, L = 4096, 7, 50
V, D = 1000000, 31
OUT = 3
DP = 32
N_ROWS = B * NF * L
NC, NS = 2, 16
NW = NC * NS
ROWS_PER_W = N_ROWS // NW
CHUNK = 128
NBUF = 4
N_GROUPS = ROWS_PER_W // (CHUNK * NBUF)
VB = 128
NVB = V // VB
VTAIL = V - NVB * VB
BLOCKS_PER_W = (NVB + NW - 1) // NW


def _transpose_block(src_v, dst_v, iota32, nv16, srow=0, dbase=0):
    for g in range(nv16):
        for d in range(D):
            val = src_v[srow + d, pl.ds(g * 16, 16)]
            plsc.store_scatter(dst_v, [iota32 + (dbase + g * 16 * DP + d)],
                               val)


def _sc_relayout_body(tt_hbm, lin_hbm, src_v, dst_v, src_t, dst_t,
                      in_sems, out_sems):
    wid = lax.axis_index("s") * NC + lax.axis_index("c")
    iota32 = lax.iota(jnp.int32, 16) * DP

    def in_copy(f, vb, buf):
        return pltpu.make_async_copy(
            tt_hbm.at[f, :, pl.ds(vb * VB, VB)],
            src_v.at[pl.ds(buf * D, D)], in_sems.at[buf])

    def out_copy(f, vb, buf):
        return pltpu.make_async_copy(
            dst_v.at[pl.ds(buf * VB * DP, VB * DP)],
            lin_hbm.at[pl.ds((f * V + vb * VB) * DP, VB * DP)],
            out_sems.at[buf])

    def step(f, k, buf):
        vb = k * NW + wid
        valid = vb < NVB

        @pl.when((k + 1) * NW + wid < NVB)
        def _():
            in_copy(f, (k + 1) * NW + wid, 1 - buf).start()

        @pl.when(valid & (k >= 2))
        def _():
            out_copy(f, (k - 2) * NW + wid, buf).wait()

        @pl.when(valid)
        def _():
            in_copy(f, vb, buf).wait()
            _transpose_block(src_v, dst_v, iota32, VB // 16,
                             srow=buf * D, dbase=buf * VB * DP)
            out_copy(f, vb, buf).start()

    P2 = 123

    def pair(q, carry):
        f = q // P2
        p = q % P2

        @pl.when((p == 0) & (wid < NVB))
        def _():
            in_copy(f, wid, 0).start()

        step(f, 2 * p, 0)
        step(f, 2 * p + 1, 1)

        @pl.when(p == P2 - 1)
        def _():
            for k in (2 * P2 - 3, 2 * P2 - 2):
                @pl.when(k * NW + wid < NVB)
                def _():
                    out_copy(f, k * NW + wid, k % 2).wait()
        return carry

    lax.fori_loop(0, NF * P2, pair, 0, unroll=False)

    @pl.when(wid < NF)
    def _():
        v0 = NVB * VB
        pltpu.sync_copy(tt_hbm.at[wid, :, pl.ds(v0, VTAIL)], src_t)
        _transpose_block(src_t, dst_t, iota32, VTAIL // 16)
        pltpu.sync_copy(dst_t,
                        lin_hbm.at[pl.ds((wid * V + v0) * DP, VTAIL * DP)])


@functools.cache
def _sc_relayout():
    return pl.kernel(
        _sc_relayout_body,
        out_type=jax.ShapeDtypeStruct((NF * V * DP,), jnp.float32),
        mesh=plsc.VectorSubcoreMesh(core_axis_name="c", subcore_axis_name="s",
                                    num_cores=NC, num_subcores=NS),
        scratch_types=[
            pltpu.VMEM((2 * D, VB), jnp.float32),
            pltpu.VMEM((2 * VB * DP,), jnp.float32),
            pltpu.VMEM((D, VTAIL), jnp.float32),
            pltpu.VMEM((VTAIL * DP,), jnp.float32),
            pltpu.SemaphoreType.DMA((2,)),
            pltpu.SemaphoreType.DMA((2,)),
        ],
        compiler_params=pltpu.CompilerParams(use_tc_tiling_on_sc=True,
                                             needs_layout_passes=False),
    )


def _sc_gather_body(table_hbm, idx_hbm, out_hbm, idx_v, rows_v, sems):
    wid = lax.axis_index("s") * NC + lax.axis_index("c")
    base = wid * ROWS_PER_W
    total = N_ROWS

    def group(g, carry):
        row0 = base + g * (CHUNK * NBUF)
        for j in range(NBUF):
            @pl.when(row0 + j * CHUNK < base + ROWS_PER_W)
            def _():
                pltpu.sync_copy(idx_hbm.at[pl.ds(row0 + j * CHUNK, CHUNK)],
                                idx_v.at[j])
                pltpu.async_copy(table_hbm.at[idx_v.at[j]], rows_v.at[j],
                                 sems.at[j])
        for j in range(NBUF):
            @pl.when(row0 + j * CHUNK < base + ROWS_PER_W)
            def _():
                pltpu.make_async_copy(table_hbm.at[idx_v.at[j]], rows_v.at[j],
                                      sems.at[j]).wait()
                pltpu.sync_copy(rows_v.at[j],
                                out_hbm.at[pl.ds(row0 + j * CHUNK, CHUNK)])
        return carry

    lax.fori_loop(0, (ROWS_PER_W + CHUNK * NBUF - 1) // (CHUNK * NBUF),
                  group, 0)


@functools.cache
def _sc_gather():
    return pl.kernel(
        _sc_gather_body,
        out_type=jax.ShapeDtypeStruct((N_ROWS, DP), jnp.float32),
        mesh=plsc.VectorSubcoreMesh(core_axis_name="c", subcore_axis_name="s",
                                    num_cores=NC, num_subcores=NS),
        scratch_types=[
            pltpu.VMEM((NBUF, CHUNK), jnp.int32),
            pltpu.VMEM((NBUF, CHUNK, DP), jnp.float32),
            pltpu.SemaphoreType.DMA((NBUF,)),
        ],
        compiler_params=pltpu.CompilerParams(use_tc_tiling_on_sc=False),
    )


def _attn_encode_body(emb_ref, w_ref, encw_ref, encb_ref, out_ref):
    i = pl.program_id(0)
    xt = jnp.tanh(emb_ref[0][:, :D])
    e = jnp.sum(xt * w_ref[0, 0], axis=1, keepdims=True)
    p = jnp.exp(e - jnp.max(e))
    a = p * (1.0 / jnp.sum(p))
    contrib = jnp.dot(xt * a, encw_ref[0],
                      preferred_element_type=jnp.float32)
    prev = jnp.where(i == 0, 0.0, out_ref[...])
    tot = prev + contrib
    is_last = i == NF * L - 1
    out_ref[...] = jnp.where(is_last,
                             jnp.maximum(tot + encb_ref[...], 0.0), tot)


def kernel(x, tables, attn_w, attn_b, enc_w, enc_b):
    del attn_b
    tt = jnp.transpose(tables, (0, 2, 1))
    lin = _sc_relayout()(tt)
    table_flat = lin.reshape(NF * V, DP)

    idx = (jnp.transpose(x, (1, 2, 0))
           + (jnp.arange(NF, dtype=jnp.int32) * V)[:, None, None])
    idx_flat = idx.reshape(N_ROWS)

    emb = _sc_gather()(table_flat, idx_flat)
    emb3 = emb.reshape(NF * L, B, DP)

    out = pl.pallas_call(
        _attn_encode_body,
        grid=(NF * L,),
        in_specs=[
            pl.BlockSpec((1, B, DP), lambda i: (i, 0, 0)),
            pl.BlockSpec((1, 1, D), lambda i: (i // L, 0, 0)),
            pl.BlockSpec((1, D, OUT), lambda i: (i // L, 0, 0)),
            pl.BlockSpec((1, OUT), lambda i: (0, 0)),
        ],
        out_specs=pl.BlockSpec((B, OUT), lambda i: (0, 0)),
        out_shape=jax.ShapeDtypeStruct((B, OUT), jnp.float32),
    )(emb3, attn_w.reshape(NF, 1, D), enc_w.reshape(NF, D, OUT),
      enc_b.reshape(1, OUT))
    return out


# speedup vs baseline: 1.6089x; 1.2164x over previous
import functools

import jax
import jax.numpy as jnp
from jax import lax
from jax.experimental import pallas as pl
from jax.experimental.pallas import tpu as pltpu
from jax.experimental.pallas import tpu_sc as plsc

B, NF, L = 4096, 7, 50
V, D = 1000000, 31
OUT = 3
DP = 32
N_ROWS = B * NF * L
NC, NS = 2, 16
NW = NC * NS
ROWS_PER_W = N_ROWS // NW
CHUNK = 128
NBUF = 4
N_GROUPS = ROWS_PER_W // (CHUNK * NBUF)
VB = 128
NVB = V // VB
VTAIL = V - NVB * VB
BLOCKS_PER_W = (NVB + NW - 1) // NW


def _transpose_block(src_v, dst_v, iota32, nv16, srow=0, dbase=0):
    for g in range(nv16):
        base = dbase + g * 16 * DP
        for d0 in range(0, D, 8):
            ds = range(d0, min(d0 + 8, D))
            vals = [src_v[srow + d, pl.ds(g * 16, 16)] for d in ds]
            for d, val in zip(ds, vals):
                plsc.store_scatter(dst_v, [iota32 + (base + d)], val)


def _sc_relayout_body(tt_hbm, lin_hbm, src_v, dst_v, src_t, dst_t,
                      in_sems, out_sems):
    wid = lax.axis_index("s") * NC + lax.axis_index("c")
    iota32 = lax.iota(jnp.int32, 16) * DP

    def in_copy(f, vb, buf):
        return pltpu.make_async_copy(
            tt_hbm.at[f, :, pl.ds(vb * VB, VB)],
            src_v.at[pl.ds(buf * D, D)], in_sems.at[buf])

    def out_copy(f, vb, buf):
        return pltpu.make_async_copy(
            dst_v.at[pl.ds(buf * VB * DP, VB * DP)],
            lin_hbm.at[pl.ds((f * V + vb * VB) * DP, VB * DP)],
            out_sems.at[buf])

    def step(f, k, buf):
        vb = k * NW + wid
        valid = vb < NVB

        @pl.when((k + 1) * NW + wid < NVB)
        def _():
            in_copy(f, (k + 1) * NW + wid, 1 - buf).start()

        @pl.when(valid & (k >= 2))
        def _():
            out_copy(f, (k - 2) * NW + wid, buf).wait()

        @pl.when(valid)
        def _():
            in_copy(f, vb, buf).wait()
            _transpose_block(src_v, dst_v, iota32, VB // 16,
                             srow=buf * D, dbase=buf * VB * DP)
            out_copy(f, vb, buf).start()

    P2 = (BLOCKS_PER_W + 1) // 2

    def pair(q, carry):
        f = q // P2
        p = q % P2

        @pl.when((p == 0) & (wid < NVB))
        def _():
            in_copy(f, wid, 0).start()

        step(f, 2 * p, 0)
        step(f, 2 * p + 1, 1)

        @pl.when(p == P2 - 1)
        def _():
            for k in (2 * P2 - 3, 2 * P2 - 2):
                @pl.when(k * NW + wid < NVB)
                def _():
                    out_copy(f, k * NW + wid, k % 2).wait()
        return carry

    lax.fori_loop(0, NF * P2, pair, 0, unroll=False)

    @pl.when(wid < NF)
    def _():
        v0 = NVB * VB
        pltpu.sync_copy(tt_hbm.at[wid, :, pl.ds(v0, VTAIL)], src_t)
        _transpose_block(src_t, dst_t, iota32, VTAIL // 16)
        pltpu.sync_copy(dst_t,
                        lin_hbm.at[pl.ds((wid * V + v0) * DP, VTAIL * DP)])


@functools.cache
def _sc_relayout():
    return pl.kernel(
        _sc_relayout_body,
        out_type=jax.ShapeDtypeStruct((NF * V * DP,), jnp.float32),
        mesh=plsc.VectorSubcoreMesh(core_axis_name="c", subcore_axis_name="s",
                                    num_cores=NC, num_subcores=NS),
        scratch_types=[
            pltpu.VMEM((2 * D, VB), jnp.float32),
            pltpu.VMEM((2 * VB * DP,), jnp.float32),
            pltpu.VMEM((D, VTAIL), jnp.float32),
            pltpu.VMEM((VTAIL * DP,), jnp.float32),
            pltpu.SemaphoreType.DMA((2,)),
            pltpu.SemaphoreType.DMA((2,)),
        ],
        compiler_params=pltpu.CompilerParams(use_tc_tiling_on_sc=True,
                                             needs_layout_passes=False),
    )


def _sc_gather_body(table_hbm, idx_hbm, out_hbm, idx_v, rows_v, sems):
    wid = lax.axis_index("s") * NC + lax.axis_index("c")
    base = wid * ROWS_PER_W
    total = N_ROWS

    def group(g, carry):
        row0 = base + g * (CHUNK * NBUF)
        for j in range(NBUF):
            @pl.when(row0 + j * CHUNK < base + ROWS_PER_W)
            def _():
                pltpu.sync_copy(idx_hbm.at[pl.ds(row0 + j * CHUNK, CHUNK)],
                                idx_v.at[j])
                pltpu.async_copy(table_hbm.at[idx_v.at[j]], rows_v.at[j],
                                 sems.at[j])
        for j in range(NBUF):
            @pl.when(row0 + j * CHUNK < base + ROWS_PER_W)
            def _():
                pltpu.make_async_copy(table_hbm.at[idx_v.at[j]], rows_v.at[j],
                                      sems.at[j]).wait()
                pltpu.sync_copy(rows_v.at[j],
                                out_hbm.at[pl.ds(row0 + j * CHUNK, CHUNK)])
        return carry

    lax.fori_loop(0, (ROWS_PER_W + CHUNK * NBUF - 1) // (CHUNK * NBUF),
                  group, 0)


@functools.cache
def _sc_gather():
    return pl.kernel(
        _sc_gather_body,
        out_type=jax.ShapeDtypeStruct((N_ROWS, DP), jnp.float32),
        mesh=plsc.VectorSubcoreMesh(core_axis_name="c", subcore_axis_name="s",
                                    num_cores=NC, num_subcores=NS),
        scratch_types=[
            pltpu.VMEM((NBUF, CHUNK), jnp.int32),
            pltpu.VMEM((NBUF, CHUNK, DP), jnp.float32),
            pltpu.SemaphoreType.DMA((NBUF,)),
        ],
        compiler_params=pltpu.CompilerParams(use_tc_tiling_on_sc=False),
    )


def _attn_encode_body(emb_ref, w_ref, encw_ref, encb_ref, out_ref):
    i = pl.program_id(0)
    xt = jnp.tanh(emb_ref[0][:, :D])
    e = jnp.sum(xt * w_ref[0, 0], axis=1, keepdims=True)
    p = jnp.exp(e - jnp.max(e))
    a = p * (1.0 / jnp.sum(p))
    contrib = jnp.dot(xt * a, encw_ref[0],
                      preferred_element_type=jnp.float32)
    prev = jnp.where(i == 0, 0.0, out_ref[...])
    tot = prev + contrib
    is_last = i == NF * L - 1
    out_ref[...] = jnp.where(is_last,
                             jnp.maximum(tot + encb_ref[...], 0.0), tot)


def kernel(x, tables, attn_w, attn_b, enc_w, enc_b):
    del attn_b
    tt = jnp.transpose(tables, (0, 2, 1))
    lin = _sc_relayout()(tt)
    table_flat = lin.reshape(NF * V, DP)

    idx = (jnp.transpose(x, (1, 2, 0))
           + (jnp.arange(NF, dtype=jnp.int32) * V)[:, None, None])
    idx_flat = idx.reshape(N_ROWS)

    emb = _sc_gather()(table_flat, idx_flat)
    emb3 = emb.reshape(NF * L, B, DP)

    out = pl.pallas_call(
        _attn_encode_body,
        grid=(NF * L,),
        in_specs=[
            pl.BlockSpec((1, B, DP), lambda i: (i, 0, 0)),
            pl.BlockSpec((1, 1, D), lambda i: (i // L, 0, 0)),
            pl.BlockSpec((1, D, OUT), lambda i: (i // L, 0, 0)),
            pl.BlockSpec((1, OUT), lambda i: (0, 0)),
        ],
        out_specs=pl.BlockSpec((B, OUT), lambda i: (0, 0)),
        out_shape=jax.ShapeDtypeStruct((B, OUT), jnp.float32),
    )(emb3, attn_w.reshape(NF, 1, D), enc_w.reshape(NF, D, OUT),
      enc_b.reshape(1, OUT))
    return out
